# baseline (speedup 1.0000x reference)
import functools
import math

import jax
import jax.numpy as jnp
from jax import lax
from jax.experimental import pallas as pl
from jax.experimental.pallas import tpu as pltpu
from jax.experimental.pallas import tpu_sc as plsc

B = 128
N = 32768
K = math.ceil(0.25 * B)
L = 16
NC = 2
NS = 16
NW = NC * NS
COLS_PER_W = N // NW
CB = 256
NBLK = COLS_PER_W // CB
NGRP = CB // L
NCH = B // K


def _sort32(a):
    n = len(a)
    k = 2
    while k <= n:
        j = k // 2
        while j >= 1:
            for i in range(n):
                l = i ^ j
                if l > i:
                    lo = jnp.minimum(a[i], a[l])
                    hi = jnp.maximum(a[i], a[l])
                    if (i & k) == 0:
                        a[i], a[l] = lo, hi
                    else:
                        a[i], a[l] = hi, lo
            j //= 2
        k *= 2


def _merge_top32(s, c):
    m = [jnp.maximum(s[i], c[31 - i]) for i in range(32)]
    for j in (16, 8, 4, 2, 1):
        for i in range(32):
            l = i ^ j
            if l > i:
                lo = jnp.minimum(m[i], m[l])
                hi = jnp.maximum(m[i], m[l])
                m[i], m[l] = lo, hi
    return m


def _sc_body(x_hbm, out_hbm, buf):
    wid = lax.axis_index("s") * NC + lax.axis_index("c")

    def blk_body(blk, carry):
        c0 = wid * COLS_PER_W + blk * CB
        pltpu.sync_copy(x_hbm.at[:, pl.ds(c0, CB)], buf)

        def grp_body(g, gcarry):
            goff = g * L

            def ldr(r):
                return buf[r, pl.ds(goff, L)]

            s = [ldr(i) for i in range(K)]
            _sort32(s)
            for ch in range(1, NCH):
                c = [ldr(ch * K + i) for i in range(K)]
                _sort32(c)
                s = _merge_top32(s, c)
            t = s[0]
            rem = (s[0] == t).astype(jnp.int32)
            for i in range(1, K):
                rem = rem + (s[i] == t).astype(jnp.int32)
            eq_seen = jnp.zeros((L,), jnp.int32)
            for r in range(B):
                v = ldr(r)
                gt = v > t
                eq = v == t
                keep = gt | (eq & (eq_seen < rem))
                eq_seen = eq_seen + eq.astype(jnp.int32)
                buf[r, pl.ds(goff, L)] = jnp.where(keep, v, jnp.float32(0.0))
            return gcarry

        lax.fori_loop(0, NGRP, grp_body, jnp.int32(0))
        pltpu.sync_copy(buf, out_hbm.at[:, pl.ds(c0, CB)])
        return carry

    lax.fori_loop(0, NBLK, blk_body, jnp.int32(0))


_mesh = plsc.VectorSubcoreMesh(core_axis_name="c", subcore_axis_name="s")


@jax.jit
def kernel(x):
    f = pl.kernel(
        _sc_body,
        out_type=jax.ShapeDtypeStruct((B, N), jnp.float32),
        mesh=_mesh,
        scratch_types=[pltpu.VMEM((B, CB), jnp.float32)],
        compiler_params=pltpu.CompilerParams(needs_layout_passes=False),
    )
    return f(x)


# speedup vs baseline: 1.0457x; 1.0457x over previous
import functools
import math

import jax
import jax.numpy as jnp
from jax import lax
from jax.experimental import pallas as pl
from jax.experimental.pallas import tpu as pltpu
from jax.experimental.pallas import tpu_sc as plsc

B = 128
N = 32768
K = math.ceil(0.25 * B)
L = 16
NC = 2
NS = 16
NW = NC * NS
COLS_PER_W = N // NW
CB = 512
NBLK = COLS_PER_W // CB
NGRP = CB // L
NCH = B // K


def _sort32(a):
    n = len(a)
    k = 2
    while k <= n:
        j = k // 2
        while j >= 1:
            for i in range(n):
                l = i ^ j
                if l > i:
                    lo = jnp.minimum(a[i], a[l])
                    hi = jnp.maximum(a[i], a[l])
                    if (i & k) == 0:
                        a[i], a[l] = lo, hi
                    else:
                        a[i], a[l] = hi, lo
            j //= 2
        k *= 2


def _merge_top32(s, c):
    m = [jnp.maximum(s[i], c[31 - i]) for i in range(32)]
    for j in (16, 8, 4, 2, 1):
        for i in range(32):
            l = i ^ j
            if l > i:
                lo = jnp.minimum(m[i], m[l])
                hi = jnp.maximum(m[i], m[l])
                m[i], m[l] = lo, hi
    return m


def _sc_body(x_hbm, out_hbm, buf):
    wid = lax.axis_index("s") * NC + lax.axis_index("c")

    def blk_body(blk, carry):
        c0 = wid * COLS_PER_W + blk * CB
        pltpu.sync_copy(x_hbm.at[:, pl.ds(c0, CB)], buf)

        def grp_body(g, gcarry):
            goff = g * L

            def ldr(r):
                return buf[r, pl.ds(goff, L)]

            s = [ldr(i) for i in range(K)]
            _sort32(s)
            for ch in range(1, NCH - 1):
                c = [ldr(ch * K + i) for i in range(K)]
                _sort32(c)
                s = _merge_top32(s, c)
            c = [ldr((NCH - 1) * K + i) for i in range(K)]
            _sort32(c)
            m = [jnp.maximum(s[i], c[31 - i]) for i in range(K)]
            mins = m
            while len(mins) > 1:
                mins = [jnp.minimum(mins[2 * i], mins[2 * i + 1])
                        for i in range(len(mins) // 2)]
            t = mins[0]
            rem = (m[0] == t).astype(jnp.int32)
            for i in range(1, K):
                rem = rem + (m[i] == t).astype(jnp.int32)
            eq_seen = jnp.zeros((L,), jnp.int32)
            for r in range(B):
                v = ldr(r)
                gt = v > t
                eq = v == t
                keep = gt | (eq & (eq_seen < rem))
                eq_seen = eq_seen + eq.astype(jnp.int32)
                buf[r, pl.ds(goff, L)] = jnp.where(keep, v, jnp.float32(0.0))
            return gcarry

        lax.fori_loop(0, NGRP, grp_body, jnp.int32(0))
        pltpu.sync_copy(buf, out_hbm.at[:, pl.ds(c0, CB)])
        return carry

    lax.fori_loop(0, NBLK, blk_body, jnp.int32(0))


_mesh = plsc.VectorSubcoreMesh(core_axis_name="c", subcore_axis_name="s")


@jax.jit
def kernel(x):
    f = pl.kernel(
        _sc_body,
        out_type=jax.ShapeDtypeStruct((B, N), jnp.float32),
        mesh=_mesh,
        scratch_types=[pltpu.VMEM((B, CB), jnp.float32)],
        compiler_params=pltpu.CompilerParams(needs_layout_passes=False),
    )
    return f(x)
